# baseline (speedup 1.0000x reference)
import functools

import jax
import jax.numpy as jnp
from jax import lax
from jax.experimental import pallas as pl
from jax.experimental.pallas import tpu as pltpu
from jax.experimental.pallas import tpu_sc as plsc

VOCAB = 1000000
EMB = 64
BATCH = 4096 * 200
NW = 32
ROWS_PER_W = BATCH // NW
CHUNK = 128
K = 4
IDX_ROWS = ROWS_PER_W // CHUNK
GROUPS = IDX_ROWS // K

_mesh = plsc.VectorSubcoreMesh(core_axis_name="c", subcore_axis_name="s")


@functools.partial(
    pl.kernel,
    mesh=_mesh,
    out_type=jax.ShapeDtypeStruct((BATCH // CHUNK, CHUNK, EMB), jnp.float32),
    scratch_types=[
        pltpu.VMEM((IDX_ROWS, CHUNK), jnp.int32),
        pltpu.VMEM((K, CHUNK, EMB), jnp.float32),
        pltpu.SemaphoreType.DMA,
    ],
    compiler_params=pltpu.CompilerParams(use_tc_tiling_on_sc=False),
)
def _emb_lookup(x_hbm, tab_hbm, out_hbm, idx_v, rows_v, gsem):
    wid = lax.axis_index("s") * 2 + lax.axis_index("c")
    base = wid * IDX_ROWS
    pltpu.sync_copy(x_hbm.at[pl.ds(base, IDX_ROWS)], idx_v)

    def body(g, carry):
        copies = [
            pltpu.async_copy(tab_hbm.at[idx_v.at[g * K + j]], rows_v.at[j], gsem)
            for j in range(K)
        ]
        for c in copies:
            c.wait()
        pltpu.sync_copy(rows_v, out_hbm.at[pl.ds(base + g * K, K)])
        return carry

    lax.fori_loop(0, GROUPS, body, 0)


def kernel(x, emb_table):
    xf = x.reshape(BATCH // CHUNK, CHUNK).astype(jnp.int32)
    out = _emb_lookup(xf, emb_table)
    return out.reshape(x.shape[0], x.shape[1], EMB)


# speedup vs baseline: 1.0237x; 1.0237x over previous
import functools

import jax
import jax.numpy as jnp
from jax import lax
from jax.experimental import pallas as pl
from jax.experimental.pallas import tpu as pltpu
from jax.experimental.pallas import tpu_sc as plsc

VOCAB = 1000000
EMB = 64
BATCH = 4096 * 200
NW = 32
ROWS_PER_W = BATCH // NW
CHUNK = 128
K = 4
IDX_ROWS = ROWS_PER_W // CHUNK
GROUPS = IDX_ROWS // K

_mesh = plsc.VectorSubcoreMesh(core_axis_name="c", subcore_axis_name="s")


@functools.partial(
    pl.kernel,
    mesh=_mesh,
    out_type=jax.ShapeDtypeStruct((BATCH // CHUNK, CHUNK, EMB), jnp.float32),
    scratch_types=[
        pltpu.VMEM((IDX_ROWS, CHUNK), jnp.int32),
        pltpu.VMEM((2, K, CHUNK, EMB), jnp.float32),
        pltpu.SemaphoreType.DMA,
        pltpu.SemaphoreType.DMA,
        pltpu.SemaphoreType.DMA,
        pltpu.SemaphoreType.DMA,
    ],
    compiler_params=pltpu.CompilerParams(use_tc_tiling_on_sc=False),
)
def _emb_lookup(x_hbm, tab_hbm, out_hbm, idx_v, rows_v, g0, g1, o0, o1):
    gsems = (g0, g1)
    osems = (o0, o1)
    wid = lax.axis_index("s") * 2 + lax.axis_index("c")
    base = wid * IDX_ROWS
    pltpu.sync_copy(x_hbm.at[pl.ds(base, IDX_ROWS)], idx_v)

    def fire(g, b):
        for j in range(K):
            pltpu.async_copy(tab_hbm.at[idx_v.at[g * K + j]], rows_v.at[b].at[j],
                             gsems[b])

    def wait_gather(b):
        pltpu.make_async_copy(out_hbm.at[pl.ds(0, K)], rows_v.at[b],
                              gsems[b]).wait()

    def start_write(g, b):
        pltpu.async_copy(rows_v.at[b], out_hbm.at[pl.ds(base + g * K, K)],
                         osems[b])

    def wait_write(b):
        pltpu.make_async_copy(rows_v.at[b], out_hbm.at[pl.ds(0, K)],
                              osems[b]).wait()

    fire(0, 0)
    fire(1, 1)
    wait_gather(0)
    start_write(0, 0)

    def body(h, carry):
        g_odd = 2 * h + 1
        wait_gather(1)
        start_write(g_odd, 1)
        wait_write(0)
        fire(g_odd + 1, 0)
        wait_gather(0)
        start_write(g_odd + 1, 0)
        wait_write(1)
        fire(g_odd + 2, 1)
        return carry

    lax.fori_loop(0, (GROUPS - 2) // 2, body, 0)

    wait_gather(1)
    start_write(GROUPS - 1, 1)
    wait_write(0)
    wait_write(1)


def kernel(x, emb_table):
    xf = x.reshape(BATCH // CHUNK, CHUNK).astype(jnp.int32)
    out = _emb_lookup(xf, emb_table)
    return out.reshape(x.shape[0], x.shape[1], EMB)
